# baseline (speedup 1.0000x reference)
import functools

import jax
import jax.numpy as jnp
from jax import lax
from jax.experimental import pallas as pl
from jax.experimental.pallas import tpu as pltpu
from jax.experimental.pallas import tpu_sc as plsc

BATCH = 4096
SEQ = 200
D = 64
B = BATCH * SEQ
NC, NS = 2, 16
NW = NC * NS
GROUP = 128
PER_W = B // NW
NGROUPS = PER_W // GROUP
K = 5
NCHUNK = NGROUPS // K
NHALF = NCHUNK // 2


def _emb_body(idx_hbm, table_hbm, out_hbm, idxslab, rowbuf, gsem0, gsem1,
              wsem0, wsem1):
    wid = lax.axis_index("s") * NC + lax.axis_index("c")
    gbase = wid * NGROUPS
    gsem = (gsem0, gsem1)
    wsem = (wsem0, wsem1)

    def fire_gathers(chunk, slot):
        for j in range(K):
            pltpu.async_copy(table_hbm.at[idxslab.at[chunk * K + j]],
                             rowbuf.at[slot, j], gsem[slot])

    def wait_gathers(chunk, slot):
        for j in range(K):
            pltpu.make_async_copy(table_hbm.at[idxslab.at[chunk * K + j]],
                                  rowbuf.at[slot, j], gsem[slot]).wait()

    def fire_write(chunk, slot):
        pltpu.async_copy(rowbuf.at[slot],
                         out_hbm.at[pl.ds(gbase + chunk * K, K)], wsem[slot])

    def wait_write(chunk, slot):
        pltpu.make_async_copy(rowbuf.at[slot],
                              out_hbm.at[pl.ds(gbase + chunk * K, K)],
                              wsem[slot]).wait()

    pltpu.sync_copy(idx_hbm.at[pl.ds(gbase, NGROUPS)], idxslab)

    fire_gathers(0, 0)

    def body(i, carry):
        a = 2 * i
        b = a + 1

        @pl.when(i > 0)
        def _():
            wait_write(a - 1, 1)
        fire_gathers(b, 1)

        wait_gathers(a, 0)
        fire_write(a, 0)

        @pl.when(i + 1 < NHALF)
        def _():
            wait_write(a, 0)
            fire_gathers(a + 2, 0)

        wait_gathers(b, 1)
        fire_write(b, 1)
        return carry

    lax.fori_loop(0, NHALF, body, 0)

    wait_write(NCHUNK - 2, 0)
    wait_write(NCHUNK - 1, 1)


@functools.partial(jax.jit, static_argnames=())
def _emb_call(idx2d, table):
    mesh = plsc.VectorSubcoreMesh(core_axis_name="c", subcore_axis_name="s")
    run = functools.partial(
        pl.kernel,
        mesh=mesh,
        compiler_params=pltpu.CompilerParams(use_tc_tiling_on_sc=False),
        out_type=jax.ShapeDtypeStruct((B // GROUP, GROUP, D), jnp.float32),
        scratch_types=[
            pltpu.VMEM((NGROUPS, GROUP), jnp.int32),
            pltpu.VMEM((2, K, GROUP, D), jnp.float32),
            pltpu.SemaphoreType.DMA,
            pltpu.SemaphoreType.DMA,
            pltpu.SemaphoreType.DMA,
            pltpu.SemaphoreType.DMA,
        ],
    )(_emb_body)
    return run(idx2d, table)


def kernel(indices, table):
    idx2d = indices.astype(jnp.int32).reshape(B // GROUP, GROUP)
    out = _emb_call(idx2d, table)
    return out.reshape(BATCH, SEQ, D)
